# baseline (speedup 1.0000x reference)
import jax
import jax.numpy as jnp
from jax.experimental import pallas as pl
from jax.experimental.pallas import tpu as pltpu

_BN = 2048
_LANES = 128


def _stage1(dist, base):
    ngroups = dist.shape[1] // _LANES
    cols = [dist[:, j * _LANES:(j + 1) * _LANES] for j in range(ngroups)]
    m = cols[0]
    for cj in cols[1:]:
        m = jnp.minimum(m, cj)
    g = jnp.full(m.shape, ngroups, dtype=jnp.int32)
    for j in reversed(range(ngroups)):
        g = jnp.where(cols[j] == m, jnp.int32(j), g)
    lane_iota = jax.lax.broadcasted_iota(jnp.int32, m.shape, 1)
    return m, g * _LANES + lane_iota + base


def _kmeans_block(x_ref, c_ref, dist_ref, idx_ref, cn_ref, cb_ref,
                  m_ref, ki_ref):
    i = pl.program_id(0)
    nsteps = pl.num_programs(0)
    k = c_ref.shape[1]
    kh = k // 2

    @pl.when(i == 0)
    def _():
        c0 = c_ref[...]
        cn_ref[...] = jnp.sum(c0 * c0, axis=0, keepdims=True)
        cb_ref[...] = (c0 * jnp.float32(-2.0)).astype(jnp.bfloat16)

    @pl.when(i > 0)
    def _():
        m = m_ref[...]
        kidx = ki_ref[...]
        mrow = jnp.min(m, axis=1, keepdims=True)
        idx_ref[...] = jnp.min(jnp.where(m == mrow, kidx, k), axis=1)

    @pl.when(i < nsteps - 1)
    def _():
        xb = x_ref[...]
        xf = xb.astype(jnp.float32)
        xn = jnp.sum(xf * xf, axis=1, keepdims=True)
        parts = []
        for h in range(2):
            sl = pl.ds(h * kh, kh)
            xc2 = jax.lax.dot_general(
                xb, cb_ref[:, sl], (((1,), (0,)), ((), ())),
                preferred_element_type=jnp.float32,
                precision=jax.lax.Precision.DEFAULT,
            )
            dist = xc2 + xn + cn_ref[:, sl]
            dist_ref[:, sl] = dist
            parts.append(_stage1(dist, h * kh))
        (m1, k1), (m2, k2) = parts
        m = jnp.minimum(m1, m2)
        kidx = jnp.where(m1 <= m2, k1, k2)
        m_ref[...] = m
        ki_ref[...] = kidx


def kernel(x, C):
    n, d = x.shape
    d2, k = C.shape
    assert d == d2
    xb = x.astype(jnp.bfloat16)
    g = n // _BN
    last = g - 1
    dist, idx = pl.pallas_call(
        _kmeans_block,
        grid=(g + 1,),
        in_specs=[
            pl.BlockSpec((_BN, d), lambda i: (jnp.minimum(i, last), 0)),

            pl.BlockSpec((d, k), lambda i: (0, 0)),
        ],
        out_specs=[
            pl.BlockSpec((_BN, k), lambda i: (jnp.minimum(i, last), 0)),
            pl.BlockSpec((_BN,), lambda i: (jnp.maximum(i - 1, 0),)),
        ],
        out_shape=[
            jax.ShapeDtypeStruct((n, k), jnp.float32),
            jax.ShapeDtypeStruct((n,), jnp.int32),
        ],
        scratch_shapes=[
            pltpu.VMEM((1, k), jnp.float32),
            pltpu.VMEM((d, k), jnp.bfloat16),
            pltpu.VMEM((_BN, _LANES), jnp.float32),
            pltpu.VMEM((_BN, _LANES), jnp.int32),
        ],
    )(xb, C)
    return (idx, dist)


# speedup vs baseline: 1.2798x; 1.2798x over previous
import jax
import jax.numpy as jnp
from jax.experimental import pallas as pl
from jax.experimental.pallas import tpu as pltpu

_BN = 512
_LANES = 128


def _stage1(dist, base):
    ngroups = dist.shape[1] // _LANES
    cols = [dist[:, j * _LANES:(j + 1) * _LANES] for j in range(ngroups)]
    m = cols[0]
    for cj in cols[1:]:
        m = jnp.minimum(m, cj)
    g = jnp.full(m.shape, ngroups, dtype=jnp.int32)
    for j in reversed(range(ngroups)):
        g = jnp.where(cols[j] == m, jnp.int32(j), g)
    lane_iota = jax.lax.broadcasted_iota(jnp.int32, m.shape, 1)
    return m, g * _LANES + lane_iota + base


def _kmeans_block(x_ref, c_ref, dist_ref, idx_ref, cn_ref, cb_ref,
                  m_ref, ki_ref):
    i = pl.program_id(0)
    nsteps = pl.num_programs(0)
    k = c_ref.shape[1]
    kh = k // 2

    @pl.when(i == 0)
    def _():
        c0 = c_ref[...]
        cn_ref[...] = jnp.sum(c0 * c0, axis=0, keepdims=True)
        cb_ref[...] = (c0 * jnp.float32(-2.0)).astype(jnp.bfloat16)

    @pl.when(i > 0)
    def _():
        m = m_ref[...]
        kidx = ki_ref[...]
        mrow = jnp.min(m, axis=1, keepdims=True)
        idx_ref[...] = jnp.min(jnp.where(m == mrow, kidx, k), axis=1)

    @pl.when(i < nsteps - 1)
    def _():
        x = x_ref[...]
        xb = x.astype(jnp.bfloat16)
        xn = jnp.sum(x * x, axis=1, keepdims=True)
        parts = []
        for h in range(2):
            sl = pl.ds(h * kh, kh)
            xc2 = jax.lax.dot_general(
                xb, cb_ref[:, sl], (((1,), (0,)), ((), ())),
                preferred_element_type=jnp.float32,
                precision=jax.lax.Precision.DEFAULT,
            )
            dist = xc2 + xn + cn_ref[:, sl]
            dist_ref[:, sl] = dist
            parts.append(_stage1(dist, h * kh))
        (m1, k1), (m2, k2) = parts
        m = jnp.minimum(m1, m2)
        kidx = jnp.where(m1 <= m2, k1, k2)
        m_ref[...] = m
        ki_ref[...] = kidx


def kernel(x, C):
    n, d = x.shape
    d2, k = C.shape
    assert d == d2
    g = n // _BN
    last = g - 1
    dist, idx = pl.pallas_call(
        _kmeans_block,
        grid=(g + 1,),
        in_specs=[
            pl.BlockSpec((_BN, d), lambda i: (jnp.minimum(i, last), 0)),
            pl.BlockSpec((d, k), lambda i: (0, 0)),
        ],
        out_specs=[
            pl.BlockSpec((_BN, k), lambda i: (jnp.minimum(i, last), 0)),
            pl.BlockSpec((_BN,), lambda i: (jnp.maximum(i - 1, 0),)),
        ],
        out_shape=[
            jax.ShapeDtypeStruct((n, k), jnp.float32),
            jax.ShapeDtypeStruct((n,), jnp.int32),
        ],
        scratch_shapes=[
            pltpu.VMEM((1, k), jnp.float32),
            pltpu.VMEM((d, k), jnp.bfloat16),
            pltpu.VMEM((_BN, _LANES), jnp.float32),
            pltpu.VMEM((_BN, _LANES), jnp.int32),
        ],
    )(x, C)
    return (idx, dist)


# speedup vs baseline: 1.3558x; 1.0593x over previous
import jax
import jax.numpy as jnp
from jax.experimental import pallas as pl
from jax.experimental.pallas import tpu as pltpu

_BN = 1024
_LANES = 128


def _stage1(dist, base):
    ngroups = dist.shape[1] // _LANES
    cols = [dist[:, j * _LANES:(j + 1) * _LANES] for j in range(ngroups)]
    m = cols[0]
    for cj in cols[1:]:
        m = jnp.minimum(m, cj)
    g = jnp.full(m.shape, ngroups, dtype=jnp.int32)
    for j in reversed(range(ngroups)):
        g = jnp.where(cols[j] == m, jnp.int32(j), g)
    lane_iota = jax.lax.broadcasted_iota(jnp.int32, m.shape, 1)
    return m, g * _LANES + lane_iota + base


def _kmeans_block(x_ref, c_ref, dist_ref, idx_ref, cn_ref, cb_ref,
                  m_ref, ki_ref):
    i = pl.program_id(0)
    nsteps = pl.num_programs(0)
    k = c_ref.shape[1]
    kh = k // 2

    @pl.when(i == 0)
    def _():
        c0 = c_ref[...]
        cn_ref[...] = jnp.sum(c0 * c0, axis=0, keepdims=True)
        cb_ref[...] = (c0 * jnp.float32(-2.0)).astype(jnp.bfloat16)

    @pl.when(i > 0)
    def _():
        m = m_ref[...]
        kidx = ki_ref[...]
        mrow = jnp.min(m, axis=1, keepdims=True)
        idx_ref[...] = jnp.min(jnp.where(m == mrow, kidx, k), axis=1)

    @pl.when(i < nsteps - 1)
    def _():
        x = x_ref[...]
        xb = x.astype(jnp.bfloat16)
        xn = jnp.sum(x * x, axis=1, keepdims=True)
        parts = []
        for h in range(2):
            sl = pl.ds(h * kh, kh)
            xc2 = jax.lax.dot_general(
                xb, cb_ref[:, sl], (((1,), (0,)), ((), ())),
                preferred_element_type=jnp.float32,
                precision=jax.lax.Precision.DEFAULT,
            )
            dist = xc2 + xn + cn_ref[:, sl]
            dist_ref[:, sl] = dist
            parts.append(_stage1(dist, h * kh))
        (m1, k1), (m2, k2) = parts
        m = jnp.minimum(m1, m2)
        kidx = jnp.where(m1 <= m2, k1, k2)
        m_ref[...] = m
        ki_ref[...] = kidx


def kernel(x, C):
    n, d = x.shape
    d2, k = C.shape
    assert d == d2
    g = n // _BN
    last = g - 1
    dist, idx = pl.pallas_call(
        _kmeans_block,
        grid=(g + 1,),
        in_specs=[
            pl.BlockSpec((_BN, d), lambda i: (jnp.minimum(i, last), 0)),
            pl.BlockSpec((d, k), lambda i: (0, 0)),
        ],
        out_specs=[
            pl.BlockSpec((_BN, k), lambda i: (jnp.minimum(i, last), 0)),
            pl.BlockSpec((_BN,), lambda i: (jnp.maximum(i - 1, 0),)),
        ],
        out_shape=[
            jax.ShapeDtypeStruct((n, k), jnp.float32),
            jax.ShapeDtypeStruct((n,), jnp.int32),
        ],
        scratch_shapes=[
            pltpu.VMEM((1, k), jnp.float32),
            pltpu.VMEM((d, k), jnp.bfloat16),
            pltpu.VMEM((_BN, _LANES), jnp.float32),
            pltpu.VMEM((_BN, _LANES), jnp.int32),
        ],
    )(x, C)
    return (idx, dist)
